# baseline (speedup 1.0000x reference)
import functools

import jax
import jax.numpy as jnp
from jax import lax
from jax.experimental import pallas as pl
from jax.experimental.pallas import tpu as pltpu
from jax.experimental.pallas import tpu_sc as plsc

_CH = 128
_HCH = 64
_NW = 32


def _sc_gather_bf16(table, idx, n_chunks, n_rows_pad, d):
    q, r = divmod(n_chunks, _NW)
    iters = q + 1
    quads = -(-iters // 4)
    mesh = plsc.VectorSubcoreMesh(core_axis_name="c", subcore_axis_name="s")

    @functools.partial(
        pl.kernel,
        mesh=mesh,
        compiler_params=pltpu.CompilerParams(needs_layout_passes=False),
        out_type=jax.ShapeDtypeStruct((n_rows_pad // 2, d), jnp.int32),
        scratch_types=[
            pltpu.VMEM((iters * _CH,), jnp.int32),
            pltpu.VMEM((4, _CH, d), jnp.float32),
            pltpu.VMEM((4, _HCH, d), jnp.int32),
            pltpu.SemaphoreType.DMA((4,)),
            pltpu.SemaphoreType.DMA((4,)),
        ],
    )
    def gather_kernel(table_hbm, idx_hbm, out_hbm, idx_all, rowsf, packb,
                      gsem, wsem):
        wid = lax.axis_index("s") * 2 + lax.axis_index("c")
        cnt = q + jnp.where(wid < r, 1, 0)
        base = wid * q + jnp.minimum(wid, r)
        pltpu.sync_copy(idx_hbm.at[pl.ds(base * _CH, iters * _CH)], idx_all)

        def start(i, s):
            @pl.when(i < cnt)
            def _():
                pltpu.async_copy(
                    table_hbm.at[idx_all.at[pl.ds(i * _CH, _CH)]],
                    rowsf.at[s], gsem.at[s])

        def conv_wb(i, s):
            @pl.when(i < cnt)
            def _():
                pltpu.make_async_copy(
                    table_hbm.at[idx_all.at[pl.ds(i * _CH, _CH)]],
                    rowsf.at[s], gsem.at[s]).wait()

                @pl.when(i >= 4)
                def _():
                    co = (base + i - 4) * _HCH
                    pltpu.make_async_copy(
                        packb.at[s], out_hbm.at[pl.ds(co, _HCH)],
                        wsem.at[s]).wait()

                @plsc.parallel_loop(0, _HCH, unroll=4)
                def conv(rr):
                    avs = [rowsf[s, 2 * rr, pl.ds(g8 * 16, 16)]
                           for g8 in range(d // 16)]
                    bvs = [rowsf[s, 2 * rr + 1, pl.ds(g8 * 16, 16)]
                           for g8 in range(d // 16)]
                    for g8 in range(d // 16):
                        p = plsc.pack(avs[g8], bvs[g8],
                                      format=plsc.PackFormat.INTERLEAVED)
                        packb[s, rr, pl.ds(g8 * 16, 16)] = plsc.bitcast(
                            p, jnp.int32)

                pltpu.async_copy(packb.at[s],
                                 out_hbm.at[pl.ds((base + i) * _HCH, _HCH)],
                                 wsem.at[s])

        start(0, 0)
        start(1, 1)
        start(2, 2)

        def body(t, carry):
            for u in range(4):
                i = 4 * t + u
                conv_wb(i, u)
                start(i + 3, (u + 3) % 4)
            return carry

        lax.fori_loop(0, quads, body, 0)

        for dd in range(4):
            i = cnt - 4 + dd

            @pl.when(i >= 0)
            def _():
                s = lax.rem(i, 4)
                co = (base + i) * _HCH
                pltpu.make_async_copy(packb.at[s],
                                      out_hbm.at[pl.ds(co, _HCH)],
                                      wsem.at[s]).wait()

    return gather_kernel(table, idx)


def _tc_mlp(packed, w1a, w1b, b1, w2, b2, w3, n, k, d, bn):
    bk = bn * k
    nblocks = n // bn
    self_block0 = (n * k) // bn

    def body(g_ref, s_ref, w1a_ref, w1b_ref, b1_ref, w2_ref, b2_ref,
             w3_ref, out_ref):
        g = pltpu.bitcast(g_ref[...], jnp.bfloat16)
        s = pltpu.bitcast(s_ref[...], jnp.bfloat16)
        u1 = jnp.dot(s, w1b_ref[...], preferred_element_type=jnp.float32)
        u1 = u1 + b1_ref[...]
        u_rep = jnp.broadcast_to(u1[:, None, :], (bn, k, d)).reshape(bk, d)
        h1 = jnp.dot(g, w1a_ref[...], preferred_element_type=jnp.float32)
        h1 = jnp.maximum(h1 + u_rep, 0.0).astype(jnp.bfloat16)
        h2 = jnp.dot(h1, w2_ref[...], preferred_element_type=jnp.float32)
        h2 = jnp.maximum(h2 + b2_ref[...], 0.0)
        t = (h2 * w3_ref[...]).reshape(bn, k, d)
        sc = jnp.sum(t, axis=2, keepdims=True)
        m = jnp.max(sc, axis=1, keepdims=True)
        e = jnp.exp(sc - m)
        att = e / jnp.sum(e, axis=1, keepdims=True)
        g3 = g.reshape(bn, k, d).astype(jnp.float32)
        agg = jnp.sum(att * g3, axis=1)
        out_ref[...] = (agg + s.astype(jnp.float32)) * 0.5

    return pl.pallas_call(
        body,
        grid=(nblocks,),
        in_specs=[
            pl.BlockSpec((bk // 2, d), lambda b: (b, 0)),
            pl.BlockSpec((bn // 2, d), lambda b: (self_block0 + b, 0)),
            pl.BlockSpec((d, d), lambda b: (0, 0)),
            pl.BlockSpec((d, d), lambda b: (0, 0)),
            pl.BlockSpec((1, d), lambda b: (0, 0)),
            pl.BlockSpec((d, d), lambda b: (0, 0)),
            pl.BlockSpec((1, d), lambda b: (0, 0)),
            pl.BlockSpec((1, d), lambda b: (0, 0)),
        ],
        out_specs=pl.BlockSpec((bn, d), lambda b: (b, 0)),
        out_shape=jax.ShapeDtypeStruct((n, d), jnp.float32),
        compiler_params=pltpu.CompilerParams(
            dimension_semantics=("arbitrary",)),
    )(packed, packed, w1a, w1b, b1, w2, b2, w3)


def kernel(nodes, neighbors, u2e_weight, W1, b1, W2, b2, W3, b3):
    n, k = neighbors.shape
    v, d = u2e_weight.shape

    w1a = W1[:, :d].T.astype(jnp.bfloat16)
    w1b = W1[:, d:].T.astype(jnp.bfloat16)
    b1r = b1.reshape(1, d)
    w2t = W2.T.astype(jnp.bfloat16)
    b2r = b2.reshape(1, d)
    w3r = W3.reshape(1, d)

    sizes = [3600, 3200, 2400, 800]
    bn = 400
    outs = []
    p0 = 0
    for pn in sizes:
        nb = neighbors[p0:p0 + pn].reshape(-1)
        nd = nodes[p0:p0 + pn]
        p0 += pn
        total = pn * k + pn
        n_chunks = -(-total // _CH)
        n_rows_pad = n_chunks * _CH
        idx = jnp.concatenate([
            nb, nd, jnp.zeros((n_rows_pad + _CH - total,), jnp.int32)])
        packed = _sc_gather_bf16(u2e_weight, idx, n_chunks, n_rows_pad, d)
        outs.append(_tc_mlp(packed, w1a, w1b, b1r, w2t, b2r, w3r,
                            pn, k, d, bn))
    return jnp.concatenate(outs, axis=0)


# speedup vs baseline: 1.0057x; 1.0057x over previous
import functools

import jax
import jax.numpy as jnp
from jax import lax
from jax.experimental import pallas as pl
from jax.experimental.pallas import tpu as pltpu
from jax.experimental.pallas import tpu_sc as plsc

_CH = 128
_HCH = 64
_NW = 32


def _sc_gather_bf16(table, idx, n_chunks, n_rows_pad, d):
    q, r = divmod(n_chunks, _NW)
    iters = q + 1
    quads = -(-iters // 4)
    mesh = plsc.VectorSubcoreMesh(core_axis_name="c", subcore_axis_name="s")

    @functools.partial(
        pl.kernel,
        mesh=mesh,
        compiler_params=pltpu.CompilerParams(needs_layout_passes=False),
        out_type=jax.ShapeDtypeStruct((n_rows_pad // 2, d), jnp.int32),
        scratch_types=[
            pltpu.VMEM((iters * _CH,), jnp.int32),
            pltpu.VMEM((4, _CH, d), jnp.float32),
            pltpu.VMEM((4, _HCH, d), jnp.int32),
            pltpu.SemaphoreType.DMA((4,)),
            pltpu.SemaphoreType.DMA((4,)),
        ],
    )
    def gather_kernel(table_hbm, idx_hbm, out_hbm, idx_all, rowsf, packb,
                      gsem, wsem):
        wid = lax.axis_index("s") * 2 + lax.axis_index("c")
        cnt = q + jnp.where(wid < r, 1, 0)
        base = wid * q + jnp.minimum(wid, r)
        pltpu.sync_copy(idx_hbm.at[pl.ds(base * _CH, iters * _CH)], idx_all)

        def start(i, s):
            @pl.when(i < cnt)
            def _():
                pltpu.async_copy(
                    table_hbm.at[idx_all.at[pl.ds(i * _CH, _CH)]],
                    rowsf.at[s], gsem.at[s])

        def conv_wb(i, s):
            @pl.when(i < cnt)
            def _():
                pltpu.make_async_copy(
                    table_hbm.at[idx_all.at[pl.ds(i * _CH, _CH)]],
                    rowsf.at[s], gsem.at[s]).wait()

                @pl.when(i >= 4)
                def _():
                    co = (base + i - 4) * _HCH
                    pltpu.make_async_copy(
                        packb.at[s], out_hbm.at[pl.ds(co, _HCH)],
                        wsem.at[s]).wait()

                @plsc.parallel_loop(0, _HCH, unroll=4)
                def conv(rr):
                    avs = [rowsf[s, 2 * rr, pl.ds(g8 * 16, 16)]
                           for g8 in range(d // 16)]
                    bvs = [rowsf[s, 2 * rr + 1, pl.ds(g8 * 16, 16)]
                           for g8 in range(d // 16)]
                    for g8 in range(d // 16):
                        p = plsc.pack(avs[g8], bvs[g8],
                                      format=plsc.PackFormat.INTERLEAVED)
                        packb[s, rr, pl.ds(g8 * 16, 16)] = plsc.bitcast(
                            p, jnp.int32)

                pltpu.async_copy(packb.at[s],
                                 out_hbm.at[pl.ds((base + i) * _HCH, _HCH)],
                                 wsem.at[s])

        start(0, 0)
        start(1, 1)
        start(2, 2)

        def body(t, carry):
            for u in range(4):
                i = 4 * t + u
                conv_wb(i, u)
                start(i + 3, (u + 3) % 4)
            return carry

        lax.fori_loop(0, quads, body, 0)

        for dd in range(4):
            i = cnt - 4 + dd

            @pl.when(i >= 0)
            def _():
                s = lax.rem(i, 4)
                co = (base + i) * _HCH
                pltpu.make_async_copy(packb.at[s],
                                      out_hbm.at[pl.ds(co, _HCH)],
                                      wsem.at[s]).wait()

    return gather_kernel(table, idx)


def _tc_mlp(packed, w1a, w1b, b1, w2, b2, w3, n, k, d, bn):
    bk = bn * k
    nblocks = n // bn
    self_block0 = (n * k) // bn

    def body(g_ref, s_ref, w1a_ref, w1b_ref, b1_ref, w2_ref, b2_ref,
             w3_ref, out_ref):
        g = pltpu.bitcast(g_ref[...], jnp.bfloat16)
        s = pltpu.bitcast(s_ref[...], jnp.bfloat16)
        u1 = jnp.dot(s, w1b_ref[...], preferred_element_type=jnp.float32)
        u1 = u1 + b1_ref[...]
        u_rep = jnp.broadcast_to(u1[:, None, :], (bn, k, d)).reshape(bk, d)
        h1 = jnp.dot(g, w1a_ref[...], preferred_element_type=jnp.float32)
        h1 = jnp.maximum(h1 + u_rep, 0.0).astype(jnp.bfloat16)
        h2 = jnp.dot(h1, w2_ref[...], preferred_element_type=jnp.float32)
        h2 = jnp.maximum(h2 + b2_ref[...], 0.0)
        t = (h2 * w3_ref[...]).reshape(bn, k, d)
        sc = jnp.sum(t, axis=2, keepdims=True)
        m = jnp.max(sc, axis=1, keepdims=True)
        e = jnp.exp(sc - m)
        att = e / jnp.sum(e, axis=1, keepdims=True)
        g3 = g.reshape(bn, k, d).astype(jnp.float32)
        agg = jnp.sum(att * g3, axis=1)
        out_ref[...] = (agg + s.astype(jnp.float32)) * 0.5

    return pl.pallas_call(
        body,
        grid=(nblocks,),
        in_specs=[
            pl.BlockSpec((bk // 2, d), lambda b: (b, 0)),
            pl.BlockSpec((bn // 2, d), lambda b: (self_block0 + b, 0)),
            pl.BlockSpec((d, d), lambda b: (0, 0)),
            pl.BlockSpec((d, d), lambda b: (0, 0)),
            pl.BlockSpec((1, d), lambda b: (0, 0)),
            pl.BlockSpec((d, d), lambda b: (0, 0)),
            pl.BlockSpec((1, d), lambda b: (0, 0)),
            pl.BlockSpec((1, d), lambda b: (0, 0)),
        ],
        out_specs=pl.BlockSpec((bn, d), lambda b: (b, 0)),
        out_shape=jax.ShapeDtypeStruct((n, d), jnp.float32),
        compiler_params=pltpu.CompilerParams(
            dimension_semantics=("arbitrary",)),
    )(packed, packed, w1a, w1b, b1, w2, b2, w3)


def kernel(nodes, neighbors, u2e_weight, W1, b1, W2, b2, W3, b3):
    n, k = neighbors.shape
    v, d = u2e_weight.shape

    w1a = W1[:, :d].T.astype(jnp.bfloat16)
    w1b = W1[:, d:].T.astype(jnp.bfloat16)
    b1r = b1.reshape(1, d)
    w2t = W2.T.astype(jnp.bfloat16)
    b2r = b2.reshape(1, d)
    w3r = W3.reshape(1, d)

    sizes = [2400, 2400, 2000, 1600, 1200, 400]
    bn = 400
    outs = []
    p0 = 0
    for pn in sizes:
        nb = neighbors[p0:p0 + pn].reshape(-1)
        nd = nodes[p0:p0 + pn]
        p0 += pn
        total = pn * k + pn
        n_chunks = -(-total // _CH)
        n_rows_pad = n_chunks * _CH
        idx = jnp.concatenate([
            nb, nd, jnp.zeros((n_rows_pad + _CH - total,), jnp.int32)])
        packed = _sc_gather_bf16(u2e_weight, idx, n_chunks, n_rows_pad, d)
        outs.append(_tc_mlp(packed, w1a, w1b, b1r, w2t, b2r, w3r,
                            pn, k, d, bn))
    return jnp.concatenate(outs, axis=0)


# speedup vs baseline: 1.0419x; 1.0360x over previous
import functools

import jax
import jax.numpy as jnp
from jax import lax
from jax.experimental import pallas as pl
from jax.experimental.pallas import tpu as pltpu
from jax.experimental.pallas import tpu_sc as plsc

_CH = 128
_HCH = 64
_NW = 32


def _sc_gather_bf16(table, idx, n_chunks, n_rows_pad, d):
    q, r = divmod(n_chunks, _NW)
    iters = q + 1
    quads = -(-iters // 4)
    mesh = plsc.VectorSubcoreMesh(core_axis_name="c", subcore_axis_name="s")

    @functools.partial(
        pl.kernel,
        mesh=mesh,
        compiler_params=pltpu.CompilerParams(needs_layout_passes=False),
        out_type=jax.ShapeDtypeStruct((n_rows_pad // 2, d), jnp.int32),
        scratch_types=[
            pltpu.VMEM((iters * _CH,), jnp.int32),
            pltpu.VMEM((4, _CH, d), jnp.float32),
            pltpu.VMEM((4, _HCH, d), jnp.int32),
            pltpu.SemaphoreType.DMA((4,)),
            pltpu.SemaphoreType.DMA((4,)),
        ],
    )
    def gather_kernel(table_hbm, idx_hbm, out_hbm, idx_all, rowsf, packb,
                      gsem, wsem):
        wid = lax.axis_index("s") * 2 + lax.axis_index("c")
        cnt = q + jnp.where(wid < r, 1, 0)
        base = wid * q + jnp.minimum(wid, r)
        pltpu.sync_copy(idx_hbm.at[pl.ds(base * _CH, iters * _CH)], idx_all)

        def start(i, s):
            @pl.when(i < cnt)
            def _():
                pltpu.async_copy(
                    table_hbm.at[idx_all.at[pl.ds(i * _CH, _CH)]],
                    rowsf.at[s], gsem.at[s])

        def conv_wb(i, s):
            @pl.when(i < cnt)
            def _():
                pltpu.make_async_copy(
                    table_hbm.at[idx_all.at[pl.ds(i * _CH, _CH)]],
                    rowsf.at[s], gsem.at[s]).wait()

                @pl.when(i >= 4)
                def _():
                    co = (base + i - 4) * _HCH
                    pltpu.make_async_copy(
                        packb.at[s], out_hbm.at[pl.ds(co, _HCH)],
                        wsem.at[s]).wait()

                @plsc.parallel_loop(0, _HCH, unroll=4)
                def conv(rr):
                    avs = [rowsf[s, 2 * rr, pl.ds(g8 * 16, 16)]
                           for g8 in range(d // 16)]
                    bvs = [rowsf[s, 2 * rr + 1, pl.ds(g8 * 16, 16)]
                           for g8 in range(d // 16)]
                    for g8 in range(d // 16):
                        p = plsc.pack(avs[g8], bvs[g8],
                                      format=plsc.PackFormat.INTERLEAVED)
                        packb[s, rr, pl.ds(g8 * 16, 16)] = plsc.bitcast(
                            p, jnp.int32)

                pltpu.async_copy(packb.at[s],
                                 out_hbm.at[pl.ds((base + i) * _HCH, _HCH)],
                                 wsem.at[s])

        start(0, 0)
        start(1, 1)
        start(2, 2)

        def body(t, carry):
            for u in range(4):
                i = 4 * t + u
                conv_wb(i, u)
                start(i + 3, (u + 3) % 4)
            return carry

        lax.fori_loop(0, quads, body, 0)

        for dd in range(4):
            i = cnt - 4 + dd

            @pl.when(i >= 0)
            def _():
                s = lax.rem(i, 4)
                co = (base + i) * _HCH
                pltpu.make_async_copy(packb.at[s],
                                      out_hbm.at[pl.ds(co, _HCH)],
                                      wsem.at[s]).wait()

    return gather_kernel(table, idx)


def _tc_mlp(packed, w1a, w1b, b1, w2, b2, w3, n, k, d, bn):
    bk = bn * k
    nblocks = n // bn
    self_block0 = (n * k) // bn

    def body(g_ref, s_ref, w1a_ref, w1b_ref, b1_ref, w2_ref, b2_ref,
             w3_ref, out_ref):
        g = pltpu.bitcast(g_ref[...], jnp.bfloat16)
        s = pltpu.bitcast(s_ref[...], jnp.bfloat16)
        u1 = jnp.dot(s, w1b_ref[...], preferred_element_type=jnp.float32)
        u1 = u1 + b1_ref[...]
        u_rep = jnp.broadcast_to(u1[:, None, :], (bn, k, d)).reshape(bk, d)
        h1 = jnp.dot(g, w1a_ref[...], preferred_element_type=jnp.float32)
        h1 = jnp.maximum(h1 + u_rep, 0.0).astype(jnp.bfloat16)
        h2 = jnp.dot(h1, w2_ref[...], preferred_element_type=jnp.float32)
        h2 = jnp.maximum(h2 + b2_ref[...], 0.0)
        t = (h2 * w3_ref[...]).reshape(bn, k, d)
        sc = jnp.sum(t, axis=2, keepdims=True)
        m = jnp.max(sc, axis=1, keepdims=True)
        e = jnp.exp(sc - m)
        att = e / jnp.sum(e, axis=1, keepdims=True)
        g3 = g.reshape(bn, k, d).astype(jnp.float32)
        agg = jnp.sum(att * g3, axis=1)
        out_ref[...] = (agg + s.astype(jnp.float32)) * 0.5

    return pl.pallas_call(
        body,
        grid=(nblocks,),
        in_specs=[
            pl.BlockSpec((bk // 2, d), lambda b: (b, 0)),
            pl.BlockSpec((bn // 2, d), lambda b: (self_block0 + b, 0)),
            pl.BlockSpec((d, d), lambda b: (0, 0)),
            pl.BlockSpec((d, d), lambda b: (0, 0)),
            pl.BlockSpec((1, d), lambda b: (0, 0)),
            pl.BlockSpec((d, d), lambda b: (0, 0)),
            pl.BlockSpec((1, d), lambda b: (0, 0)),
            pl.BlockSpec((1, d), lambda b: (0, 0)),
        ],
        out_specs=pl.BlockSpec((bn, d), lambda b: (b, 0)),
        out_shape=jax.ShapeDtypeStruct((n, d), jnp.float32),
        compiler_params=pltpu.CompilerParams(
            dimension_semantics=("arbitrary",)),
    )(packed, packed, w1a, w1b, b1, w2, b2, w3)


def kernel(nodes, neighbors, u2e_weight, W1, b1, W2, b2, W3, b3):
    n, k = neighbors.shape
    v, d = u2e_weight.shape

    w1a = W1[:, :d].T.astype(jnp.bfloat16)
    w1b = W1[:, d:].T.astype(jnp.bfloat16)
    b1r = b1.reshape(1, d)
    w2t = W2.T.astype(jnp.bfloat16)
    b2r = b2.reshape(1, d)
    w3r = W3.reshape(1, d)

    sizes = [3200, 2800, 2000, 1600, 400]
    bn = 400
    outs = []
    p0 = 0
    for pn in sizes:
        nb = neighbors[p0:p0 + pn].reshape(-1)
        nd = nodes[p0:p0 + pn]
        p0 += pn
        total = pn * k + pn
        n_chunks = -(-total // _CH)
        n_rows_pad = n_chunks * _CH
        idx = jnp.concatenate([
            nb, nd, jnp.zeros((n_rows_pad + _CH - total,), jnp.int32)])
        packed = _sc_gather_bf16(u2e_weight, idx, n_chunks, n_rows_pad, d)
        outs.append(_tc_mlp(packed, w1a, w1b, b1r, w2t, b2r, w3r,
                            pn, k, d, bn))
    return jnp.concatenate(outs, axis=0)
